# baseline (speedup 1.0000x reference)
import functools

import jax
import jax.numpy as jnp
import numpy as np
from jax import lax
from jax.experimental import pallas as pl
from jax.experimental.pallas import tpu as pltpu
from jax.experimental.pallas import tpu_sc as plsc

_B, _H, _D = 16384, 200, 8
_NW = 32
_BT = _B // 128
_HT = _H // 8
_TPW = _BT // _NW
_HG = 5
_NIT = _TPW * (_HT // _HG)
assert _NIT % 2 == 0 and _HT % _HG == 0

_TOKLIST = np.array([0, 0, 16384, 0, 2048, 8192, 18432, 24576,
                     2560, 10240, 18944, 26624, 2688, 10752, 19072, 27136,
                     2720, 10880, 19104, 27264, 2728, 10912, 19112, 27296,
                     2730, 10920, 19114, 27304, 10922, 10922, 10923, 27306],
                    dtype=np.int32)
_COLLIST = np.array([0] * 28 + [7, 0, 7, 0], dtype=np.int32)
_FLATIDX = ((_TOKLIST >> 7) * 1024 + _COLLIST * 128 + (_TOKLIST & 127))


@functools.partial(
    pl.kernel,
    out_type=jax.ShapeDtypeStruct((_H, _BT, _D, 128), jnp.float32),
    mesh=plsc.VectorSubcoreMesh(core_axis_name="c", subcore_axis_name="s"),
    compiler_params=pltpu.CompilerParams(needs_layout_passes=False,
                                         use_tc_tiling_on_sc=False,
                                         disable_bounds_checks=True),
    scratch_types=[
        pltpu.VMEM((32,), jnp.int32),
        pltpu.VMEM((32,), jnp.float32),
        pltpu.VMEM((_HG, 8, 128), jnp.int32),
        pltpu.VMEM((_HG, 8, 128), jnp.int32),
        pltpu.VMEM((_HG * 8, _D, 128), jnp.float32),
        pltpu.VMEM((_HG * 8, _D, 128), jnp.float32),
        pltpu.SemaphoreType.DMA,
        pltpu.SemaphoreType.DMA,
        pltpu.SemaphoreType.DMA,
        pltpu.SemaphoreType.DMA,
    ],
)
def _constellation_sc(tok_hbm, flatidx_hbm, const_hbm, out_hbm,
                      idx32_v, tab_v, tok0, tok1, out0, out1,
                      si0, si1, so0, so1):
    wid = lax.axis_index("s") * 2 + lax.axis_index("c")
    btbase = wid * _TPW

    pltpu.sync_copy(flatidx_hbm, idx32_v)
    pltpu.async_copy(const_hbm.at[idx32_v], tab_v, si0).wait()
    iota = lax.iota(jnp.int32, 16)

    tokbufs, outbufs = (tok0, tok1), (out0, out1)
    sis, sos = (si0, si1), (so0, so1)

    def decode(c):
        btl = (c * 52429) >> 18
        hg = c - btl * _HG
        return btbase + btl, hg * _HG

    def in_copy(c, tokbuf, si):
        bt, ht0 = decode(c)
        return pltpu.make_async_copy(
            tok_hbm.at[pl.ds(ht0, _HG), bt], tokbuf, si)

    def out_copy(c, outbuf, so):
        bt, ht0 = decode(c)
        return pltpu.make_async_copy(
            outbuf, out_hbm.at[pl.ds(ht0 * 8, _HG * 8), bt], so)

    def compute(tokbuf, outbuf):
        @plsc.parallel_loop(0, _HG * 8 * 8, unroll=8)
        def vec_body(v):
            hl = v >> 6
            hi = (v >> 3) & 7
            bi0 = (v & 7) * 16
            t16 = tokbuf[hl, hi, pl.ds(bi0, 16)] << 1
            x = t16 & 0x5555
            s = (x & 0x1111) + ((x >> 2) & 0x1111)
            key = ((s * 0x1111) >> 10) & 0x3C
            hrow = hl * 8 + hi
            for c in range(8):
                sh = 14 - 4 * (c & 3) - 2 * (c >> 2)
                p = (t16 >> sh) & 3
                val = plsc.load_gather(tab_v, [key | p])
                outbuf[hrow, c, pl.ds(bi0, 16)] = val

    in_copy(0, tok0, si0).start()
    in_copy(1, tok1, si1).start()

    def body(g, carry):
        for b in range(2):
            c = 2 * g + b
            tokbuf, outbuf, si, so = tokbufs[b], outbufs[b], sis[b], sos[b]
            in_copy(c, tokbuf, si).wait()

            @pl.when(g > 0)
            def _wait_out():
                out_copy(c, outbuf, so).wait()

            compute(tokbuf, outbuf)
            out_copy(c, outbuf, so).start()

            @pl.when(c + 2 < _NIT)
            def _prefetch():
                in_copy(c + 2, tokbuf, si).start()
        return carry

    lax.fori_loop(0, _NIT // 2, body, 0)
    out_copy(_NIT - 2, out0, so0).wait()
    out_copy(_NIT - 1, out1, so1).wait()


def kernel(token_ids, const_real):
    tok4 = token_ids.reshape(_BT, 128, _HT, 8).transpose(2, 0, 3, 1)
    constp = const_real.reshape(256, 128, 8).transpose(0, 2, 1).reshape(-1)
    out4 = _constellation_sc(tok4, jnp.asarray(_FLATIDX), constp)
    return out4.transpose(1, 3, 0, 2).reshape(_B, _H, _D)


# speedup vs baseline: 1.0114x; 1.0114x over previous
import functools

import jax
import jax.numpy as jnp
import numpy as np
from jax import lax
from jax.experimental import pallas as pl
from jax.experimental.pallas import tpu as pltpu
from jax.experimental.pallas import tpu_sc as plsc

_B, _H, _D = 16384, 200, 8
_NW = 32
_BT = _B // 128
_HT = _H // 8
_TPW = _BT // _NW
_HG = 5
_NIT = _TPW * (_HT // _HG)
assert _NIT % 2 == 0 and _HT % _HG == 0

_TOKLIST = np.array([0, 0, 16384, 0, 2048, 8192, 18432, 24576,
                     2560, 10240, 18944, 26624, 2688, 10752, 19072, 27136,
                     2720, 10880, 19104, 27264, 2728, 10912, 19112, 27296,
                     2730, 10920, 19114, 27304, 10922, 10922, 10923, 27306],
                    dtype=np.int32)
_COLLIST = np.array([0] * 28 + [7, 0, 7, 0], dtype=np.int32)
_FLATIDX = ((_TOKLIST >> 7) * 1024 + _COLLIST * 128 + (_TOKLIST & 127))


@functools.partial(
    pl.kernel,
    out_type=jax.ShapeDtypeStruct((_H, _BT, _D, 128), jnp.float32),
    mesh=plsc.VectorSubcoreMesh(core_axis_name="c", subcore_axis_name="s"),
    compiler_params=pltpu.CompilerParams(needs_layout_passes=False,
                                         use_tc_tiling_on_sc=False,
                                         disable_bounds_checks=True),
    scratch_types=[
        pltpu.VMEM((32,), jnp.int32),
        pltpu.VMEM((32,), jnp.float32),
        pltpu.VMEM((_HG, 8, 128), jnp.int32),
        pltpu.VMEM((_HG, 8, 128), jnp.int32),
        pltpu.VMEM((_HG * 8, _D, 128), jnp.float32),
        pltpu.VMEM((_HG * 8, _D, 128), jnp.float32),
        pltpu.SemaphoreType.DMA,
        pltpu.SemaphoreType.DMA,
        pltpu.SemaphoreType.DMA,
        pltpu.SemaphoreType.DMA,
    ],
)
def _constellation_sc(tok_hbm, flatidx_hbm, const_hbm, out_hbm,
                      idx32_v, tab_v, tok0, tok1, out0, out1,
                      si0, si1, so0, so1):
    wid = lax.axis_index("s") * 2 + lax.axis_index("c")
    btbase = wid * _TPW

    pltpu.sync_copy(flatidx_hbm, idx32_v)
    pltpu.async_copy(const_hbm.at[idx32_v], tab_v, si0).wait()

    tokbufs, outbufs = (tok0, tok1), (out0, out1)
    sis, sos = (si0, si1), (so0, so1)

    def decode(c):
        btl = (c * 52429) >> 18
        hg = c - btl * _HG
        return btbase + btl, hg * _HG

    def in_copy(c, tokbuf, si):
        bt, ht0 = decode(c)
        return pltpu.make_async_copy(
            tok_hbm.at[pl.ds(ht0, _HG), bt], tokbuf, si)

    def out_copy(c, outbuf, so):
        bt, ht0 = decode(c)
        return pltpu.make_async_copy(
            outbuf, out_hbm.at[pl.ds(ht0 * 8, _HG * 8), bt], so)

    def compute(tokbuf, outbuf):
        @plsc.parallel_loop(0, _HG * 8 * 8, unroll=4)
        def vec_body(v):
            hl = v >> 6
            hi = (v >> 3) & 7
            bi0 = (v & 7) * 16
            t16 = tokbuf[hl, hi, pl.ds(bi0, 16)] << 1
            x = t16 & 0x5555
            s = (x & 0x1111) + ((x >> 2) & 0x1111)
            key = ((s * 0x1111) >> 10) & 0x3C
            hrow = hl * 8 + hi
            for c in range(8):
                sh = 14 - 4 * (c & 3) - 2 * (c >> 2)
                p = (t16 >> sh) & 3
                val = plsc.load_gather(tab_v, [key | p])
                outbuf[hrow, c, pl.ds(bi0, 16)] = val

    in_copy(0, tok0, si0).start()
    in_copy(1, tok1, si1).start()

    def body(g, carry):
        for b in range(2):
            c = 2 * g + b
            tokbuf, outbuf, si, so = tokbufs[b], outbufs[b], sis[b], sos[b]
            in_copy(c, tokbuf, si).wait()

            @pl.when(g > 0)
            def _wait_out():
                out_copy(c, outbuf, so).wait()

            compute(tokbuf, outbuf)
            out_copy(c, outbuf, so).start()

            @pl.when(c + 2 < _NIT)
            def _prefetch():
                in_copy(c + 2, tokbuf, si).start()
        return carry

    lax.fori_loop(0, _NIT // 2, body, 0)
    out_copy(_NIT - 2, out0, so0).wait()
    out_copy(_NIT - 1, out1, so1).wait()


def kernel(token_ids, const_real):
    tok4 = token_ids.reshape(_BT, 128, _HT, 8).transpose(2, 0, 3, 1)
    constp = const_real.reshape(256, 128, 8).transpose(0, 2, 1).reshape(-1)
    out4 = _constellation_sc(tok4, jnp.asarray(_FLATIDX), constp)
    return out4.transpose(1, 3, 0, 2).reshape(_B, _H, _D)
